# baseline (speedup 1.0000x reference)
import jax
import jax.numpy as jnp
from jax.experimental import pallas as pl

S = 512
BLOCK = 256


def _net_kernel(x_ref, wv1_ref, wv2_ref, wq1_ref, bq1_ref, wq2_ref, bq2_ref,
                wvh1_ref, bvh1_ref, wvh2_ref, bvh2_ref, out_ref):
    s = x_ref[...]
    h = jnp.maximum(jnp.dot(s, wv1_ref[...], preferred_element_type=jnp.float32), 0.0)
    v = jnp.dot(h, wv2_ref[...], preferred_element_type=jnp.float32)
    hq = jnp.maximum(
        jnp.dot(v, wq1_ref[...], preferred_element_type=jnp.float32) + bq1_ref[...], 0.0)
    q = jnp.dot(hq, wq2_ref[...], preferred_element_type=jnp.float32) + bq2_ref[...]
    hv = jnp.maximum(
        jnp.dot(v, wvh1_ref[...], preferred_element_type=jnp.float32) + bvh1_ref[...], 0.0)
    vs = jnp.sum(hv * wvh2_ref[...], axis=1, keepdims=True) + bvh2_ref[...]
    out_ref[...] = q - jnp.mean(q, axis=1, keepdims=True) + vs


def kernel(x, Wq1, bq1, Wq2, bq2, Wk1, Wk2, Wv1, Wv2,
           WQ1, bQ1, WQ2, bQ2, WV1, bV1, WV2, bV2):
    ba = x.shape[0]
    wv1 = Wv1.T
    wv2 = Wv2.T
    wq1 = WQ1.T
    wq2 = WQ2.T
    wvh1 = WV1.T
    wvh2 = WV2
    bq1_ = bQ1.reshape(1, -1)
    bq2_ = bQ2.reshape(1, -1)
    bvh1_ = bV1.reshape(1, -1)
    bvh2_ = bV2.reshape(1, -1)

    out_dim = WQ2.shape[0]
    grid = (ba // BLOCK,)

    def full(a):
        return pl.BlockSpec(a.shape, lambda i: (0,) * a.ndim)

    return pl.pallas_call(
        _net_kernel,
        grid=grid,
        in_specs=[
            pl.BlockSpec((BLOCK, S), lambda i: (i, 0)),
            full(wv1), full(wv2), full(wq1), full(bq1_), full(wq2), full(bq2_),
            full(wvh1), full(bvh1_), full(wvh2), full(bvh2_),
        ],
        out_specs=pl.BlockSpec((BLOCK, out_dim), lambda i: (i, 0)),
        out_shape=jax.ShapeDtypeStruct((ba, out_dim), jnp.float32),
    )(x, wv1, wv2, wq1, bq1_, wq2, bq2_, wvh1, bvh1_, wvh2, bvh2_)


# speedup vs baseline: 1.1685x; 1.1685x over previous
import jax
import jax.numpy as jnp
from jax.experimental import pallas as pl
from jax.experimental.pallas import tpu as pltpu

S = 512
BLOCK = 512


def _net_kernel(x_ref, wv1_ref, wv2_ref, wq1_ref, bq1_ref, wq2_ref, bq2_ref,
                wvh1_ref, bvh1_ref, wvh2_ref, bvh2_ref, out_ref):
    s = x_ref[...]
    h = jnp.maximum(jnp.dot(s, wv1_ref[...], preferred_element_type=jnp.float32), 0.0)
    v = jnp.dot(h, wv2_ref[...], preferred_element_type=jnp.float32)
    hq = jnp.maximum(
        jnp.dot(v, wq1_ref[...], preferred_element_type=jnp.float32) + bq1_ref[...], 0.0)
    q = jnp.dot(hq, wq2_ref[...], preferred_element_type=jnp.float32) + bq2_ref[...]
    hv = jnp.maximum(
        jnp.dot(v, wvh1_ref[...], preferred_element_type=jnp.float32) + bvh1_ref[...], 0.0)
    vs = jnp.sum(hv * wvh2_ref[...], axis=1, keepdims=True) + bvh2_ref[...]
    out_ref[...] = q - jnp.mean(q, axis=1, keepdims=True) + vs


def kernel(x, Wq1, bq1, Wq2, bq2, Wk1, Wk2, Wv1, Wv2,
           WQ1, bQ1, WQ2, bQ2, WV1, bV1, WV2, bV2):
    ba = x.shape[0]
    wv1 = Wv1.T
    wv2 = Wv2.T
    wq1 = WQ1.T
    wq2 = WQ2.T
    wvh1 = WV1.T
    wvh2 = WV2
    bq1_ = bQ1.reshape(1, -1)
    bq2_ = bQ2.reshape(1, -1)
    bvh1_ = bV1.reshape(1, -1)
    bvh2_ = bV2.reshape(1, -1)

    out_dim = WQ2.shape[0]
    grid = (ba // BLOCK,)

    def full(a):
        return pl.BlockSpec(a.shape, lambda i: (0,) * a.ndim)

    return pl.pallas_call(
        _net_kernel,
        grid=grid,
        in_specs=[
            pl.BlockSpec((BLOCK, S), lambda i: (i, 0)),
            full(wv1), full(wv2), full(wq1), full(bq1_), full(wq2), full(bq2_),
            full(wvh1), full(bvh1_), full(wvh2), full(bvh2_),
        ],
        out_specs=pl.BlockSpec((BLOCK, out_dim), lambda i: (i, 0)),
        out_shape=jax.ShapeDtypeStruct((ba, out_dim), jnp.float32),
        compiler_params=pltpu.CompilerParams(
            dimension_semantics=("parallel",)),
    )(x, wv1, wv2, wq1, bq1_, wq2, bq2_, wvh1, bvh1_, wvh2, bvh2_)
